# baseline (speedup 1.0000x reference)
import functools

import jax
import jax.numpy as jnp
from jax import lax
from jax.experimental import pallas as pl
from jax.experimental.pallas import tpu as pltpu
from jax.experimental.pallas import tpu_sc as plsc

_VOCAB = 100000
_D = 64
_B = 16384
_NEG = 3
_NEG_PAD = 8

_NC, _NS = 2, 16
_NW = _NC * _NS
_BPW = _B // _NW

@functools.cache
def _build_sc_gather():
    mesh = plsc.VectorSubcoreMesh(core_axis_name="c", subcore_axis_name="s")

    @functools.partial(
        pl.kernel,
        mesh=mesh,
        compiler_params=pltpu.CompilerParams(use_tc_tiling_on_sc=False),
        out_type=[
            jax.ShapeDtypeStruct((_B, _D), jnp.float32),
            jax.ShapeDtypeStruct((_B, _D), jnp.float32),
            jax.ShapeDtypeStruct((_NEG_PAD, _D), jnp.float32),
        ],
        scratch_types=[
            pltpu.VMEM((_BPW,), jnp.int32),
            pltpu.VMEM((_BPW, _D), jnp.float32),
            pltpu.VMEM((_BPW,), jnp.int32),
            pltpu.VMEM((_BPW, _D), jnp.float32),
            pltpu.VMEM((_NEG_PAD,), jnp.int32),
            pltpu.VMEM((_NEG_PAD, _D), jnp.float32),
            pltpu.SemaphoreType.DMA,
            pltpu.SemaphoreType.DMA,
            pltpu.SemaphoreType.DMA,
        ],
    )
    def _sc_gather(emb_hbm, tidx_hbm, cidx_hbm, nidx_hbm, t_out, c_out, n_out,
                   tiv, trv, civ, crv, niv, nrv, sem_t, sem_c, sem_n):
        wid = lax.axis_index("s") * _NC + lax.axis_index("c")
        base = wid * _BPW
        pltpu.sync_copy(tidx_hbm.at[pl.ds(base, _BPW)], tiv)
        pltpu.sync_copy(cidx_hbm.at[pl.ds(base, _BPW)], civ)
        cp_t = pltpu.async_copy(emb_hbm.at[tiv], trv, sem_t)
        cp_c = pltpu.async_copy(emb_hbm.at[civ], crv, sem_c)
        cp_t.wait()
        pltpu.sync_copy(trv, t_out.at[pl.ds(base, _BPW)])
        cp_c.wait()
        pltpu.sync_copy(crv, c_out.at[pl.ds(base, _BPW)])

        @pl.when(wid == 0)
        def _():
            pltpu.sync_copy(nidx_hbm, niv)
            pltpu.async_copy(emb_hbm.at[niv], nrv, sem_n).wait()
            pltpu.sync_copy(nrv, n_out)

    return _sc_gather


_BLK = 2048


def _log_sigmoid(x):
    return jnp.minimum(x, 0.0) - jnp.log1p(jnp.exp(-jnp.abs(x)))


def _tc_body(t_ref, c_ref, n_ref, out_ref):
    i = pl.program_id(0)
    t = t_ref[...]
    c = c_ref[...]
    n = n_ref[...]
    pos = jnp.sum(t * c, axis=1, keepdims=True)
    negd = lax.dot_general(t, n, (((1,), (1,)), ((), ())),
                           preferred_element_type=jnp.float32)
    mask = lax.broadcasted_iota(jnp.int32, (_BLK, _NEG_PAD), 1) < _NEG
    neg_ls = jnp.where(mask, _log_sigmoid(-negd), 0.0)
    total = jnp.sum(_log_sigmoid(pos)) + jnp.sum(neg_ls)

    @pl.when(i == 0)
    def _():
        out_ref[0, 0] = 0.0

    out_ref[0, 0] += total


def _tc_loss(t_rows, c_rows, n_rows):
    return pl.pallas_call(
        _tc_body,
        grid=(_B // _BLK,),
        in_specs=[
            pl.BlockSpec((_BLK, _D), lambda i: (i, 0)),
            pl.BlockSpec((_BLK, _D), lambda i: (i, 0)),
            pl.BlockSpec((_NEG_PAD, _D), lambda i: (0, 0)),
        ],
        out_specs=pl.BlockSpec(memory_space=pltpu.SMEM),
        out_shape=jax.ShapeDtypeStruct((1, 1), jnp.float32),
    )(t_rows, c_rows, n_rows)


def kernel(target_idx, context_idx, embeddings, neg_idx):
    nidx = jnp.concatenate(
        [neg_idx.astype(jnp.int32),
         jnp.zeros((_NEG_PAD - _NEG,), jnp.int32)])
    t_rows, c_rows, n_rows = _build_sc_gather()(
        embeddings, target_idx.astype(jnp.int32), context_idx.astype(jnp.int32),
        nidx)
    acc = _tc_loss(t_rows, c_rows, n_rows)
    return -acc[0, 0] / _B
